# baseline (speedup 1.0000x reference)
import functools

import jax
import jax.numpy as jnp
from jax import lax
from jax.experimental import pallas as pl
from jax.experimental.pallas import tpu as pltpu
from jax.experimental.pallas import tpu_sc as plsc

_C = 128
_NBUF = 4
_AHEAD = 2
_K = 2


def _make_piece_kernel(N, Vp, W):
    info = plsc.get_sparse_core_info()
    NC, NS = info.num_cores, info.num_subcores
    NW = NC * NS
    per_w = N // NW
    n_chunks = per_w // _C
    n_groups = n_chunks // _NBUF
    assert n_chunks % _NBUF == 0 and per_w % _C == 0 and N % NW == 0
    mesh = plsc.VectorSubcoreMesh(core_axis_name="c", subcore_axis_name="s")

    @functools.partial(
        pl.kernel,
        mesh=mesh,
        scratch_types=[
            pltpu.VMEM((n_chunks, _C), jnp.int32),
            pltpu.VMEM((n_chunks, _C), jnp.int32),
            pltpu.VMEM((_NBUF, _C, W), jnp.float32),
            pltpu.SemaphoreType.DMA((_NBUF,)),
            pltpu.SemaphoreType.DMA((_NBUF,)),
        ],
        compiler_params=pltpu.CompilerParams(use_tc_tiling_on_sc=False),
    )
    def piece_kernel(
        table_hbm, idx_hbm, orow_hbm, out_hbm, idx_v, orow_v, rows_v, gsem,
        osem,
    ):
        wid = lax.axis_index("s") * NC + lax.axis_index("c")
        chunk0 = wid * n_chunks

        pltpu.sync_copy(idx_hbm.at[pl.ds(chunk0, n_chunks)], idx_v)
        pltpu.sync_copy(orow_hbm.at[pl.ds(chunk0, n_chunks)], orow_v)

        def gather_op(local_j, buf):
            src = table_hbm.at[
                plsc.Indices(idx_v.at[local_j], ignored_value=-1)
            ]
            return pltpu.make_async_copy(src, rows_v.at[buf], gsem.at[buf])

        def scatter_op(local_j, buf):
            dst = out_hbm.at[
                plsc.Indices(orow_v.at[local_j], ignored_value=-1)
            ]
            return pltpu.make_async_copy(rows_v.at[buf], dst, osem.at[buf])

        for p in range(_AHEAD):
            gather_op(p, p).start()

        def group_body(g, carry):
            j0 = g * _NBUF
            for p in range(_NBUF):
                j = j0 + p
                gather_op(j, p).wait()
                scatter_op(j, p).start()
                q = (p + _AHEAD) % _NBUF
                jn = j + _AHEAD

                @pl.when(j >= _NBUF - _AHEAD)
                def _():
                    scatter_op(j - (_NBUF - _AHEAD), q).wait()

                @pl.when(jn < n_chunks)
                def _():
                    gather_op(jn, q).start()

            return carry

        lax.fori_loop(0, n_groups, group_body, 0)

        for p in range(_NBUF - _AHEAD):
            j = n_chunks - (_NBUF - _AHEAD) + p
            scatter_op(j, j % _NBUF).wait()

    return piece_kernel


def kernel(x, table):
    B, S = x.shape
    V, D = table.shape
    N = B * S
    W = 128
    Vp = V // _K
    assert V % _K == 0
    xi = x.reshape(N // _C, _C).astype(jnp.int32)
    rowid = jnp.arange(N, dtype=jnp.int32).reshape(N // _C, _C)
    out_ref = jax.new_ref(jnp.zeros((N, W), jnp.float32))
    piece = _make_piece_kernel(N, Vp, W)
    for k in range(_K):
        table_p = jnp.pad(table[k * Vp : (k + 1) * Vp], ((0, 0), (0, W - D)))
        in_piece = (xi >= k * Vp) & (xi < (k + 1) * Vp)
        idx_k = jnp.where(in_piece, xi - k * Vp, -1)
        orow_k = jnp.where(in_piece, rowid, -1)
        piece(table_p, idx_k, orow_k, out_ref)
    out = out_ref[...]
    return out[:, :D].reshape(B, S, D)


# speedup vs baseline: 1.5235x; 1.5235x over previous
import functools

import jax
import jax.numpy as jnp
from jax import lax
from jax.experimental import pallas as pl
from jax.experimental.pallas import tpu as pltpu
from jax.experimental.pallas import tpu_sc as plsc
_C = 128
_NBUF = 5
_AHEAD = 3


def _make_gather(N, V, D):
    info = plsc.get_sparse_core_info()
    NC, NS = info.num_cores, info.num_subcores
    NW = NC * NS
    per_w = N // NW
    n_chunks = per_w // _C
    n_groups = n_chunks // _NBUF
    assert n_chunks % _NBUF == 0 and per_w % _C == 0 and N % NW == 0
    mesh = plsc.VectorSubcoreMesh(core_axis_name="c", subcore_axis_name="s")

    @functools.partial(
        pl.kernel,
        mesh=mesh,
        out_type=jax.ShapeDtypeStruct((N, D), jnp.float32),
        scratch_types=[
            pltpu.VMEM((n_chunks, _C), jnp.int32),
            pltpu.VMEM((_NBUF, _C, D), jnp.float32),
            pltpu.SemaphoreType.DMA((_NBUF,)),
            pltpu.SemaphoreType.DMA((_NBUF,)),
        ],
        compiler_params=pltpu.CompilerParams(use_tc_tiling_on_sc=False),
    )
    def gather_kernel(table_hbm, idx_hbm, out_hbm, idx_v, rows_v, gsem, osem):
        wid = lax.axis_index("s") * NC + lax.axis_index("c")
        chunk0 = wid * n_chunks

        pltpu.sync_copy(idx_hbm.at[pl.ds(chunk0, n_chunks)], idx_v)

        def issue_gather(local_j, buf):
            pltpu.async_copy(
                table_hbm.at[idx_v.at[local_j]], rows_v.at[buf], gsem.at[buf]
            )

        def issue_out(local_j, buf):
            pltpu.async_copy(
                rows_v.at[buf, :, pl.ds(0, D // 2)],
                out_hbm.at[pl.ds((chunk0 + local_j) * _C, _C), pl.ds(0, D // 2)],
                osem.at[buf],
            )

        for p in range(_AHEAD):
            issue_gather(p, p)

        def group_body(g, carry):
            j0 = g * _NBUF
            for p in range(_NBUF):
                j = j0 + p
                pltpu.make_async_copy(
                    table_hbm.at[idx_v.at[j]], rows_v.at[p], gsem.at[p]
                ).wait()
                issue_out(j, p)
                q = (p + _AHEAD) % _NBUF
                jn = j + _AHEAD

                @pl.when(j >= _NBUF - _AHEAD)
                def _():
                    pltpu.make_async_copy(
                        rows_v.at[q, :, pl.ds(0, D // 2)],
                        out_hbm.at[
                            pl.ds((chunk0 + j - (_NBUF - _AHEAD)) * _C, _C),
                            pl.ds(0, D // 2),
                        ],
                        osem.at[q],
                    ).wait()

                @pl.when(jn < n_chunks)
                def _():
                    issue_gather(jn, q)

            return carry

        lax.fori_loop(0, n_groups, group_body, 0)

        for p in range(_NBUF - _AHEAD):
            j = n_chunks - (_NBUF - _AHEAD) + p
            pltpu.make_async_copy(
                rows_v.at[j % _NBUF, :, pl.ds(0, D // 2)],
                out_hbm.at[pl.ds((chunk0 + j) * _C, _C), pl.ds(0, D // 2)],
                osem.at[j % _NBUF],
            ).wait()

    return gather_kernel


def kernel(x, table):
    B, S = x.shape
    V, D = table.shape
    N = B * S
    table_p = jnp.pad(table, ((0, 0), (0, 128 - D)))
    flat_idx = x.reshape(N // _C, _C).astype(jnp.int32)
    out = _make_gather(N, V, 128)(table_p, flat_idx)
    return out[:, :D].reshape(B, S, D)
